# Initial kernel scaffold; baseline (speedup 1.0000x reference)
import jax
import jax.numpy as jnp
from jax.experimental import pallas as pl


def kernel(x, table):
    raise NotImplementedError("write your pallas kernel here")


# speedup vs baseline: 3.7983x; 3.7983x over previous
import functools
import math

import jax
import jax.numpy as jnp
from jax import lax
from jax.experimental import pallas as pl
from jax.experimental.pallas import tpu as pltpu
from jax.experimental.pallas import tpu_sc as plsc

D_MODEL = 64
SCALE = math.sqrt(D_MODEL)

IDXW = 128
CHUNK = 512
KSUB = CHUNK // IDXW


def _scale_body(t_ref, o_ref):
    o_ref[...] = t_ref[...] * SCALE


@functools.lru_cache(maxsize=None)
def _make_scale(v, d):
    blk = 2000
    assert v % blk == 0
    return pl.pallas_call(
        _scale_body,
        grid=(v // blk,),
        in_specs=[pl.BlockSpec((blk, d), lambda i: (i, 0))],
        out_specs=pl.BlockSpec((blk, d), lambda i: (i, 0)),
        out_shape=jax.ShapeDtypeStruct((v, d), jnp.float32),
    )


@functools.lru_cache(maxsize=None)
def _make_gather(b, v, d):
    info = plsc.get_sparse_core_info()
    nc, ns = info.num_cores, info.num_subcores
    nw = nc * ns
    assert b % (nw * CHUNK) == 0
    b_per_w = b // nw
    g_total = b_per_w // CHUNK
    assert g_total % 2 == 0 and g_total >= 4
    idx_rows_per_chunk = CHUNK // IDXW
    idx_rows_per_w = b_per_w // IDXW

    mesh = plsc.VectorSubcoreMesh(core_axis_name="c", subcore_axis_name="s")

    @functools.partial(
        pl.kernel,
        mesh=mesh,
        compiler_params=pltpu.CompilerParams(use_tc_tiling_on_sc=False),
        out_type=jax.ShapeDtypeStruct((b, d), jnp.float32),
        scratch_types=[
            pltpu.VMEM((2, KSUB, IDXW), jnp.int32),
            pltpu.VMEM((2, CHUNK, d), jnp.float32),
            pltpu.SemaphoreType.DMA,
            pltpu.SemaphoreType.DMA,
            pltpu.SemaphoreType.DMA,
            pltpu.SemaphoreType.DMA,
        ],
    )
    def gather_kernel(idx_hbm, tab_hbm, out_hbm, idx_v, rows_v, gsem0, gsem1,
                      osem0, osem1):
        gsems = (gsem0, gsem1)
        osems = (osem0, osem1)
        wid = lax.axis_index("s") * nc + lax.axis_index("c")
        row_base = wid * b_per_w
        idx_row_base = wid * idx_rows_per_w

        def load_idx(g, bslot):
            pltpu.sync_copy(
                idx_hbm.at[pl.ds(idx_row_base + g * idx_rows_per_chunk,
                                 idx_rows_per_chunk)],
                idx_v.at[bslot],
            )

        def start_gather(bslot):
            for j in range(KSUB):
                pltpu.async_copy(
                    tab_hbm.at[idx_v.at[bslot, j]],
                    rows_v.at[bslot, pl.ds(j * IDXW, IDXW)],
                    gsems[bslot],
                )

        def wait_gather(bslot):
            for j in range(KSUB):
                pltpu.make_async_copy(
                    tab_hbm.at[idx_v.at[bslot, j]],
                    rows_v.at[bslot, pl.ds(j * IDXW, IDXW)],
                    gsems[bslot],
                ).wait()

        def start_write(g, bslot):
            pltpu.async_copy(
                rows_v.at[bslot],
                out_hbm.at[pl.ds(row_base + g * CHUNK, CHUNK)],
                osems[bslot],
            )

        def wait_write(g, bslot):
            pltpu.make_async_copy(
                rows_v.at[bslot],
                out_hbm.at[pl.ds(row_base + g * CHUNK, CHUNK)],
                osems[bslot],
            ).wait()

        for bslot in range(2):
            load_idx(bslot, bslot)
            start_gather(bslot)

        def body(g2, carry):
            for bslot in range(2):
                g = g2 * 2 + bslot
                wait_gather(bslot)
                start_write(g, bslot)
                wait_write(g, bslot)
                load_idx(g + 2, bslot)
                start_gather(bslot)
            return carry

        lax.fori_loop(0, g_total // 2 - 1, body, 0, unroll=False)

        for bslot in range(2):
            g = g_total - 2 + bslot
            wait_gather(bslot)
            start_write(g, bslot)
            wait_write(g, bslot)

    return gather_kernel


def kernel(x, table):
    v, d = table.shape
    orig_shape = x.shape
    b = x.size
    tab_scaled = _make_scale(v, d)(table)
    idx2d = x.reshape(b // IDXW, IDXW).astype(jnp.int32)
    out = _make_gather(b, v, d)(idx2d, tab_scaled)
    return out.reshape(*orig_shape, d)


# speedup vs baseline: 3.8509x; 1.0138x over previous
import functools
import math

import jax
import jax.numpy as jnp
from jax import lax
from jax.experimental import pallas as pl
from jax.experimental.pallas import tpu as pltpu
from jax.experimental.pallas import tpu_sc as plsc

D_MODEL = 64
SCALE = math.sqrt(D_MODEL)

IDXW = 128
CHUNK = 256
KSUB = CHUNK // IDXW
NBUF = 6
LAG = 3


def _scale_body(t_ref, o_ref):
    o_ref[...] = t_ref[...] * SCALE


@functools.lru_cache(maxsize=None)
def _make_scale(v, d):
    blk = 2000
    assert v % blk == 0
    return pl.pallas_call(
        _scale_body,
        grid=(v // blk,),
        in_specs=[pl.BlockSpec((blk, d), lambda i: (i, 0))],
        out_specs=pl.BlockSpec((blk, d), lambda i: (i, 0)),
        out_shape=jax.ShapeDtypeStruct((v, d), jnp.float32),
    )


@functools.lru_cache(maxsize=None)
def _make_gather(b, v, d):
    info = plsc.get_sparse_core_info()
    nc, ns = info.num_cores, info.num_subcores
    nw = nc * ns
    assert b % (nw * CHUNK) == 0
    b_per_w = b // nw
    g_total = b_per_w // CHUNK
    assert g_total > NBUF
    idx_rows_per_w = b_per_w // IDXW

    mesh = plsc.VectorSubcoreMesh(core_axis_name="c", subcore_axis_name="s")

    @functools.partial(
        pl.kernel,
        mesh=mesh,
        compiler_params=pltpu.CompilerParams(use_tc_tiling_on_sc=False),
        out_type=jax.ShapeDtypeStruct((b, d), jnp.float32),
        scratch_types=[
            pltpu.VMEM((idx_rows_per_w, IDXW), jnp.int32),
            pltpu.VMEM((NBUF, CHUNK, d), jnp.float32),
        ]
        + [pltpu.SemaphoreType.DMA] * (2 * NBUF),
    )
    def gather_kernel(idx_hbm, tab_hbm, out_hbm, idx_v, rows_v, *sems):
        gsems = sems[:NBUF]
        osems = sems[NBUF:]
        wid = lax.axis_index("s") * nc + lax.axis_index("c")
        row_base = wid * b_per_w
        idx_row_base = wid * idx_rows_per_w

        pltpu.sync_copy(
            idx_hbm.at[pl.ds(idx_row_base, idx_rows_per_w)], idx_v
        )

        def gather_descs(g):
            bslot = g % NBUF
            return [
                (
                    tab_hbm.at[idx_v.at[g * KSUB + j]],
                    rows_v.at[bslot, pl.ds(j * IDXW, IDXW)],
                    gsems[bslot],
                )
                for j in range(KSUB)
            ]

        def write_desc(g):
            bslot = g % NBUF
            return (
                rows_v.at[bslot],
                out_hbm.at[pl.ds(row_base + g * CHUNK, CHUNK)],
                osems[bslot],
            )

        for g in range(g_total + LAG):
            if g < g_total:
                if g >= NBUF:
                    pltpu.make_async_copy(*write_desc(g - NBUF)).wait()
                for desc in gather_descs(g):
                    pltpu.async_copy(*desc)
            gp = g - LAG
            if gp >= 0:
                for desc in gather_descs(gp):
                    pltpu.make_async_copy(*desc).wait()
                pltpu.async_copy(*write_desc(gp))

        for g in range(g_total + LAG - NBUF, g_total):
            pltpu.make_async_copy(*write_desc(g)).wait()

    return gather_kernel


def kernel(x, table):
    v, d = table.shape
    orig_shape = x.shape
    b = x.size
    tab_scaled = _make_scale(v, d)(table)
    idx2d = x.reshape(b // IDXW, IDXW).astype(jnp.int32)
    out = _make_gather(b, v, d)(idx2d, tab_scaled)
    return out.reshape(*orig_shape, d)
